# baseline (speedup 1.0000x reference)
import functools

import jax
import jax.numpy as jnp
from jax import lax
from jax.experimental import pallas as pl
from jax.experimental.pallas import tpu as pltpu
from jax.experimental.pallas import tpu_sc as plsc

_D = 32
_NC = 2
_NS = 16
_NW = _NC * _NS


@functools.partial(jax.jit, static_argnames=())
def _gather_rows(idx, table):
    total = idx.shape[0]
    per_w = total // _NW
    chunk = 1600
    n_chunks = per_w // chunk

    mesh = plsc.VectorSubcoreMesh(core_axis_name="c", subcore_axis_name="s")

    @functools.partial(
        pl.kernel,
        mesh=mesh,
        out_type=jax.ShapeDtypeStruct((total, _D), jnp.float32),
        scratch_types=[
            pltpu.VMEM((per_w,), jnp.int32),
            pltpu.VMEM((chunk, _D), jnp.float32),
            pltpu.SemaphoreType.DMA,
        ],
        compiler_params=pltpu.CompilerParams(use_tc_tiling_on_sc=False),
    )
    def k(idx_hbm, table_hbm, out_hbm, idx_v, rows_v, sem):
        wid = lax.axis_index("s") * _NC + lax.axis_index("c")
        base = wid * per_w
        pltpu.sync_copy(idx_hbm.at[pl.ds(base, per_w)], idx_v)

        def body(j, carry):
            off = j * chunk
            pltpu.async_copy(
                table_hbm.at[idx_v.at[pl.ds(off, chunk)]], rows_v, sem
            ).wait()
            pltpu.sync_copy(rows_v, out_hbm.at[pl.ds(base + off, chunk)])
            return carry

        lax.fori_loop(0, n_chunks, body, 0)

    return k(idx, table)


def kernel(user, memory, item_table, user_table):
    b, n = memory.shape
    idx = memory.reshape(b * n).astype(jnp.int32)
    out = _gather_rows(idx, item_table)
    return out.reshape(b, n * _D)


# speedup vs baseline: 1.0010x; 1.0010x over previous
import functools

import jax
import jax.numpy as jnp
from jax import lax
from jax.experimental import pallas as pl
from jax.experimental.pallas import tpu as pltpu
from jax.experimental.pallas import tpu_sc as plsc

_D = 32
_NC = 2
_NS = 16
_NW = _NC * _NS


@functools.partial(jax.jit, static_argnames=())
def _gather_rows(idx, table):
    total = idx.shape[0]
    per_w = total // _NW
    chunk = 800
    n_chunks = per_w // chunk
    nbuf = 4

    mesh = plsc.VectorSubcoreMesh(core_axis_name="c", subcore_axis_name="s")

    @functools.partial(
        pl.kernel,
        mesh=mesh,
        out_type=jax.ShapeDtypeStruct((total, _D), jnp.float32),
        scratch_types=[
            pltpu.VMEM((per_w,), jnp.int32),
            [pltpu.VMEM((chunk, _D), jnp.float32) for _ in range(nbuf)],
            [pltpu.SemaphoreType.DMA for _ in range(nbuf)],
            [pltpu.SemaphoreType.DMA for _ in range(nbuf)],
        ],
        compiler_params=pltpu.CompilerParams(use_tc_tiling_on_sc=False),
    )
    def k(idx_hbm, table_hbm, out_hbm, idx_v, bufs, gsems, wsems):
        wid = lax.axis_index("s") * _NC + lax.axis_index("c")
        base = wid * per_w
        pltpu.sync_copy(idx_hbm.at[pl.ds(base, per_w)], idx_v)

        def gather(j):
            b = j % nbuf
            return pltpu.async_copy(
                table_hbm.at[idx_v.at[pl.ds(j * chunk, chunk)]], bufs[b], gsems[b]
            )

        def writeout(j):
            b = j % nbuf
            return pltpu.async_copy(
                bufs[b], out_hbm.at[pl.ds(base + j * chunk, chunk)], wsems[b]
            )

        hg = [None] * n_chunks
        hw = [None] * n_chunks
        hg[0] = gather(0)
        for j in range(n_chunks):
            if j + 1 < n_chunks:
                if j + 1 >= nbuf:
                    hw[j + 1 - nbuf].wait()
                hg[j + 1] = gather(j + 1)
            hg[j].wait()
            hw[j] = writeout(j)
        for j in range(max(0, n_chunks - nbuf), n_chunks):
            hw[j].wait()

    return k(idx, table)


def kernel(user, memory, item_table, user_table):
    b, n = memory.shape
    idx = memory.reshape(b * n).astype(jnp.int32)
    out = _gather_rows(idx, item_table)
    return out.reshape(b, n * _D)
